# baseline (speedup 1.0000x reference)
import jax
import jax.numpy as jnp
from jax import lax
from jax.experimental import pallas as pl
from jax.experimental.pallas import tpu as pltpu
from jax.experimental.pallas import tpu_sc as plsc

M = 1000000
B = 16384
D = 64
L = 16
NC = 2
NS = 16
MPAD = 1 << 20
RPT = MPAD // NS
JBITS = 14
SPW = B // (NC * NS)
CH = 128
NCH = SPW // CH
INVALID = 0x7FFFFFFF


def _body(memT_hbm, idx_hbm, valT_hbm, samp_hbm, outT_hbm,
          idx_v, samp_v, tag_v, t_v, tc_v, mf_v, g2_v, v2_v, tag_hbm,
          gsem, vsem):
    cid = lax.axis_index("c")
    sid = lax.axis_index("s")
    lanes = lax.iota(jnp.int32, L)
    shift = jnp.minimum(lanes + 1, L - 1)

    pltpu.sync_copy(idx_hbm, idx_v)

    base_row = sid * RPT

    def scan_body(k, _):
        x = idx_v[pl.ds(k * L, L)]
        jv = k * L + lanes
        local = x - base_row
        valid = (local >= 0) & (local < RPT)
        comp = jnp.where(valid, (local << JBITS) | jv, INVALID)
        comp_s, _unused_vals = plsc.sort_key_val(comp, comp)
        loc_s = lax.shift_right_arithmetic(comp_s, JBITS)
        j_s = comp_s & (B - 1)
        valid_s = comp_s < (1 << (JBITS + 16))
        nxt = comp_s.at[shift].get(mode="promise_in_bounds")
        nxt_loc = lax.shift_right_arithmetic(nxt, JBITS)
        win = valid_s & ((loc_s != nxt_loc) | (lanes == L - 1))
        loc_c = jnp.minimum(loc_s, RPT - 1)
        plsc.store_scatter(tag_v, [loc_c], j_s, mask=win)
        return _

    lax.fori_loop(0, B // L, scan_body, None)

    pltpu.sync_copy(tag_v, tag_hbm.at[pl.ds(cid * MPAD + sid * RPT, RPT)])
    plsc.subcore_barrier()

    base_s = (cid * NS + sid) * SPW
    pltpu.sync_copy(samp_hbm.at[pl.ds(base_s, SPW)], samp_v)
    tag_half = tag_hbm.at[pl.ds(cid * MPAD, MPAD)]

    def chunk_body(c, _):
        sl = samp_v.at[pl.ds(c * CH, CH)]
        pltpu.sync_copy(tag_half.at[sl], t_v)

        def mask_body(i, _):
            t = t_v[pl.ds(i * L, L)]
            s = samp_v[pl.ds(c * CH + i * L, L)]
            inb = (t >= 0) & (t < B)
            tc = jnp.where(inb, t, 0)
            back = plsc.load_gather(idx_v, [tc])
            hit = inb & (back == s)
            tc_v[pl.ds(i * L, L)] = tc
            mf_v[pl.ds(i * L, L)] = jnp.where(hit, 1.0, 0.0).astype(jnp.float32)
            return _

        lax.fori_loop(0, CH // L, mask_body, None)

        gd = [pltpu.async_copy(memT_hbm.at[f].at[sl], g2_v.at[f], gsem)
              for f in range(D)]
        vd = [pltpu.async_copy(valT_hbm.at[f].at[tc_v], v2_v.at[f], vsem)
              for f in range(D)]
        for d in gd:
            d.wait()
        for d in vd:
            d.wait()

        def blend_body(i, _):
            mf = mf_v[pl.ds((i % (CH // L)) * L, L)]
            f = i // (CH // L)
            g = g2_v[f, pl.ds((i % (CH // L)) * L, L)]
            v = v2_v[f, pl.ds((i % (CH // L)) * L, L)]
            g2_v[f, pl.ds((i % (CH // L)) * L, L)] = g + mf * (v - g)
            return _

        lax.fori_loop(0, D * (CH // L), blend_body, None)
        od = [pltpu.async_copy(
                  g2_v.at[f], outT_hbm.at[f, pl.ds(base_s + c * CH, CH)], gsem)
              for f in range(D)]
        for d in od:
            d.wait()
        return _

    lax.fori_loop(0, NCH, chunk_body, None)


@jax.jit
def kernel(mem, idx, val, sample_idx):
    mesh = plsc.VectorSubcoreMesh(
        core_axis_name="c", subcore_axis_name="s",
        num_cores=NC, num_subcores=NS)
    run = pl.kernel(
        _body,
        out_type=jax.ShapeDtypeStruct((D, B), jnp.float32),
        mesh=mesh,
        scratch_types=[
            pltpu.VMEM((B,), jnp.int32),
            pltpu.VMEM((SPW,), jnp.int32),
            pltpu.VMEM((RPT,), jnp.int32),
            pltpu.VMEM((CH,), jnp.int32),
            pltpu.VMEM((CH,), jnp.int32),
            pltpu.VMEM((CH,), jnp.float32),
            pltpu.VMEM((D, CH), jnp.float32),
            pltpu.VMEM((D, CH), jnp.float32),
            pltpu.HBM((NC * MPAD,), jnp.int32),
            pltpu.SemaphoreType.DMA,
            pltpu.SemaphoreType.DMA,
        ],
        compiler_params=pltpu.CompilerParams(
            needs_layout_passes=False, use_tc_tiling_on_sc=False),
    )
    return run(mem.T, idx, val.T, sample_idx).T


# speedup vs baseline: 4.1128x; 4.1128x over previous
import jax
import jax.numpy as jnp
from jax import lax
from jax.experimental import pallas as pl
from jax.experimental.pallas import tpu as pltpu
from jax.experimental.pallas import tpu_sc as plsc

M = 1000000
B = 16384
D = 64
L = 16
NC = 2
NS = 16
MPAD = 1 << 20
RPT = MPAD // NS
JBITS = 14
SPW = B // (NC * NS)
CH = 128
NCH = SPW // CH
INVALID = 0x7FFFFFFF


def _body(mem2_hbm, idx_hbm, val2_hbm, samp_hbm, out2_hbm,
          idx_v, samp_v, tag_v, t_v, sh_v, tv_v, hg_v, hv_v, mf_v,
          g2_v, v2_v, ob_v, tag_hbm, gsem, vsem):
    cid = lax.axis_index("c")
    sid = lax.axis_index("s")
    lanes = lax.iota(jnp.int32, L)
    shift = jnp.minimum(lanes + 1, L - 1)

    pltpu.sync_copy(idx_hbm, idx_v)

    base_row = sid * RPT

    def scan_body(k, _):
        x = idx_v[pl.ds(k * L, L)]
        jv = k * L + lanes
        local = x - base_row
        valid = (local >= 0) & (local < RPT)
        comp = jnp.where(valid, (local << JBITS) | jv, INVALID)
        comp_s, _unused_vals = plsc.sort_key_val(comp, comp)
        loc_s = lax.shift_right_arithmetic(comp_s, JBITS)
        j_s = comp_s & (B - 1)
        valid_s = comp_s < (1 << (JBITS + 16))
        nxt = comp_s.at[shift].get(mode="promise_in_bounds")
        nxt_loc = lax.shift_right_arithmetic(nxt, JBITS)
        win = valid_s & ((loc_s != nxt_loc) | (lanes == L - 1))
        loc_c = jnp.minimum(loc_s, RPT - 1)
        plsc.store_scatter(tag_v, [loc_c], j_s, mask=win)
        return _

    lax.fori_loop(0, B // L, scan_body, None)

    pltpu.sync_copy(tag_v, tag_hbm.at[pl.ds(cid * MPAD + sid * RPT, RPT)])
    plsc.subcore_barrier()

    base_s = (cid * NS + sid) * SPW
    pltpu.sync_copy(samp_hbm.at[pl.ds(base_s, SPW)], samp_v)
    tag_half = tag_hbm.at[pl.ds(cid * MPAD, MPAD)]

    def chunk_body(c, _):
        sl = samp_v.at[pl.ds(c * CH, CH)]
        pltpu.sync_copy(tag_half.at[sl], t_v)

        def mask_body(i, _):
            t = t_v[pl.ds(i * L, L)]
            s = samp_v[pl.ds(c * CH + i * L, L)]
            inb = (t >= 0) & (t < B)
            tc = jnp.where(inb, t, 0)
            back = plsc.load_gather(idx_v, [tc])
            hit = inb & (back == s)
            sh_v[pl.ds(i * L, L)] = lax.shift_right_logical(s, 1)
            tv_v[pl.ds(i * L, L)] = lax.shift_right_logical(tc, 1)
            hg_v[pl.ds(i * L, L)] = (s & 1) * D
            hv_v[pl.ds(i * L, L)] = (tc & 1) * D
            mf_v[pl.ds(i * L, L)] = jnp.where(hit, 1.0, 0.0).astype(jnp.float32)
            return _

        lax.fori_loop(0, CH // L, mask_body, None)

        gd = pltpu.async_copy(mem2_hbm.at[sh_v], g2_v, gsem)
        vd = pltpu.async_copy(val2_hbm.at[tv_v], v2_v, vsem)
        gd.wait()
        vd.wait()

        def blend_body(i, _):
            row = jnp.full((L,), i, jnp.int32)
            orow = jnp.full((L,), i // 2, jnp.int32)
            mrow = plsc.load_gather(mf_v, [row])
            hg = plsc.load_gather(hg_v, [row])
            hv = plsc.load_gather(hv_v, [row])
            for cc in range(D // L):
                g = plsc.load_gather(g2_v, [row, hg + cc * L + lanes])
                v = plsc.load_gather(v2_v, [row, hv + cc * L + lanes])
                plsc.store_scatter(
                    ob_v, [orow, (i % 2) * D + cc * L + lanes],
                    g + mrow * (v - g))
            return _

        lax.fori_loop(0, CH, blend_body, None)
        oOff = pl.multiple_of((base_s + c * CH) // 2, CH // 2)
        pltpu.sync_copy(ob_v, out2_hbm.at[pl.ds(oOff, CH // 2)])
        return _

    lax.fori_loop(0, NCH, chunk_body, None)


@jax.jit
def kernel(mem, idx, val, sample_idx):
    mesh = plsc.VectorSubcoreMesh(
        core_axis_name="c", subcore_axis_name="s",
        num_cores=NC, num_subcores=NS)
    run = pl.kernel(
        _body,
        out_type=jax.ShapeDtypeStruct((B // 2, 2 * D), jnp.float32),
        mesh=mesh,
        scratch_types=[
            pltpu.VMEM((B,), jnp.int32),
            pltpu.VMEM((SPW,), jnp.int32),
            pltpu.VMEM((RPT,), jnp.int32),
            pltpu.VMEM((CH,), jnp.int32),
            pltpu.VMEM((CH,), jnp.int32),
            pltpu.VMEM((CH,), jnp.int32),
            pltpu.VMEM((CH,), jnp.int32),
            pltpu.VMEM((CH,), jnp.int32),
            pltpu.VMEM((CH,), jnp.float32),
            pltpu.VMEM((CH, 2 * D), jnp.float32),
            pltpu.VMEM((CH, 2 * D), jnp.float32),
            pltpu.VMEM((CH // 2, 2 * D), jnp.float32),
            pltpu.HBM((NC * MPAD,), jnp.int32),
            pltpu.SemaphoreType.DMA,
            pltpu.SemaphoreType.DMA,
        ],
        compiler_params=pltpu.CompilerParams(
            needs_layout_passes=False, use_tc_tiling_on_sc=True),
    )
    out2 = run(mem.reshape(M // 2, 2 * D), idx,
               val.reshape(B // 2, 2 * D), sample_idx)
    return out2.reshape(B, D)


# speedup vs baseline: 5.3994x; 1.3128x over previous
import jax
import jax.numpy as jnp
from jax import lax
from jax.experimental import pallas as pl
from jax.experimental.pallas import tpu as pltpu
from jax.experimental.pallas import tpu_sc as plsc

M = 1000000
B = 16384
D = 64
L = 16
NC = 2
NS = 16
MPAD = 1 << 20
RPT = MPAD // NS
JBITS = 14
SPW = B // (NC * NS)
CH = 128
NCH = SPW // CH
UNROLL = 1
INVALID = 0x7FFFFFFF


def _body(mem_hbm, idx_hbm, val_hbm, samp_hbm, out_hbm,
          idx_v, samp_v, tag_v, t_v, tc_v, mf_v, g_v, v_v, tag_hbm,
          gsem, vsem):
    cid = lax.axis_index("c")
    sid = lax.axis_index("s")
    lanes = lax.iota(jnp.int32, L)
    shift = jnp.minimum(lanes + 1, L - 1)

    pltpu.sync_copy(idx_hbm, idx_v)

    base_row = sid * RPT

    def scan_one(k):
        x = idx_v[pl.ds(k * L, L)]
        jv = k * L + lanes
        local = x - base_row
        valid = (local >= 0) & (local < RPT)
        comp = jnp.where(valid, (local << JBITS) | jv, INVALID)
        comp_s, _unused = plsc.sort_key_val(comp, comp)
        loc_s = lax.shift_right_arithmetic(comp_s, JBITS)
        j_s = comp_s & (B - 1)
        valid_s = comp_s < (1 << (JBITS + 16))
        nxt = comp_s.at[shift].get(mode="promise_in_bounds")
        nxt_loc = lax.shift_right_arithmetic(nxt, JBITS)
        win = valid_s & ((loc_s != nxt_loc) | (lanes == L - 1))
        loc_c = jnp.minimum(loc_s, RPT - 1)
        return loc_c, j_s, win

    def scan_body(k4, _):
        results = [scan_one(k4 * UNROLL + u) for u in range(UNROLL)]
        for loc_c, j_s, win in results:
            plsc.store_scatter(tag_v, [loc_c], j_s, mask=win)
        return _

    lax.fori_loop(0, B // L // UNROLL, scan_body, None)

    pltpu.sync_copy(tag_v, tag_hbm.at[pl.ds(cid * MPAD + sid * RPT, RPT)])
    plsc.subcore_barrier()

    base_s = (cid * NS + sid) * SPW
    pltpu.sync_copy(samp_hbm.at[pl.ds(base_s, SPW)], samp_v)
    tag_half = tag_hbm.at[pl.ds(cid * MPAD, MPAD)]

    def chunk_body(c, _):
        sl = samp_v.at[pl.ds(c * CH, CH)]
        pltpu.sync_copy(tag_half.at[sl], t_v)

        def mask_body(i, _):
            t = t_v[pl.ds(i * L, L)]
            s = samp_v[pl.ds(c * CH + i * L, L)]
            inb = (t >= 0) & (t < B)
            tc = jnp.where(inb, t, 0)
            back = plsc.load_gather(idx_v, [tc])
            hit = inb & (back == s)
            tc_v[pl.ds(i * L, L)] = tc
            mf_v[pl.ds(i * L, L)] = jnp.where(hit, 1.0, 0.0).astype(jnp.float32)
            return _

        lax.fori_loop(0, CH // L, mask_body, None)
        gd = pltpu.async_copy(mem_hbm.at[sl], g_v, gsem)
        vd = pltpu.async_copy(val_hbm.at[tc_v], v_v, vsem)
        gd.wait()
        vd.wait()

        def row_body(r, _):
            mrow = plsc.load_gather(mf_v, [jnp.full((L,), r, jnp.int32)])
            for cc in range(D // L):
                g = g_v[r, pl.ds(cc * L, L)]
                v = v_v[r, pl.ds(cc * L, L)]
                g_v[r, pl.ds(cc * L, L)] = g + mrow * (v - g)
            return _

        lax.fori_loop(0, CH, row_body, None)
        pltpu.sync_copy(g_v, out_hbm.at[pl.ds(base_s + c * CH, CH)])
        return _

    lax.fori_loop(0, NCH, chunk_body, None)


@jax.jit
def kernel(mem, idx, val, sample_idx):
    mesh = plsc.VectorSubcoreMesh(
        core_axis_name="c", subcore_axis_name="s",
        num_cores=NC, num_subcores=NS)
    run = pl.kernel(
        _body,
        out_type=jax.ShapeDtypeStruct((B, D), jnp.float32),
        mesh=mesh,
        scratch_types=[
            pltpu.VMEM((B,), jnp.int32),
            pltpu.VMEM((SPW,), jnp.int32),
            pltpu.VMEM((RPT,), jnp.int32),
            pltpu.VMEM((CH,), jnp.int32),
            pltpu.VMEM((CH,), jnp.int32),
            pltpu.VMEM((CH,), jnp.float32),
            pltpu.VMEM((CH, D), jnp.float32),
            pltpu.VMEM((CH, D), jnp.float32),
            pltpu.HBM((NC * MPAD,), jnp.int32),
            pltpu.SemaphoreType.DMA,
            pltpu.SemaphoreType.DMA,
        ],
        compiler_params=pltpu.CompilerParams(
            needs_layout_passes=False, use_tc_tiling_on_sc=False),
    )
    return run(mem, idx, val, sample_idx)


# speedup vs baseline: 5.7131x; 1.0581x over previous
import jax
import jax.numpy as jnp
from jax import lax
from jax.experimental import pallas as pl
from jax.experimental.pallas import tpu as pltpu
from jax.experimental.pallas import tpu_sc as plsc

M = 1000000
B = 16384
D = 64
L = 16
NC = 2
NS = 16
NW = NC * NS
MPAD = 1 << 20
RPT = MPAD // NW
JBITS = 14
SPW = B // NW
CH = 128
NCH = SPW // CH
INVALID = 0x7FFFFFFF


def _tag_body(idx_hbm, tag_hbm, idx_v, tag_v):
    cid = lax.axis_index("c")
    sid = lax.axis_index("s")
    wid = cid * NS + sid
    lanes = lax.iota(jnp.int32, L)
    shift = jnp.minimum(lanes + 1, L - 1)

    pltpu.sync_copy(idx_hbm, idx_v)
    base_row = wid * RPT

    def scan_body(k, _):
        x = idx_v[pl.ds(k * L, L)]
        jv = k * L + lanes
        local = x - base_row
        valid = (local >= 0) & (local < RPT)
        comp = jnp.where(valid, (local << (JBITS + 1)) | jv, INVALID)
        comp_s, _unused = plsc.sort_key_val(comp, comp)
        loc_s = lax.shift_right_arithmetic(comp_s, JBITS + 1)
        j_s = comp_s & (B - 1)
        valid_s = comp_s < (1 << (JBITS + 16))
        nxt = comp_s.at[shift].get(mode="promise_in_bounds")
        nxt_loc = lax.shift_right_arithmetic(nxt, JBITS + 1)
        win = valid_s & ((loc_s != nxt_loc) | (lanes == L - 1))
        loc_c = jnp.minimum(loc_s, RPT - 1)
        plsc.store_scatter(tag_v, [loc_c], j_s, mask=win)
        return _

    lax.fori_loop(0, B // L, scan_body, None)
    pltpu.sync_copy(tag_v, tag_hbm.at[pl.ds(wid * RPT, RPT)])


def _out_body(mem_hbm, idx_hbm, val_hbm, samp_hbm, tag_hbm, out_hbm,
              idx_v, samp_v, t_v, tc_v, mf_v, g_v, v_v, gsem, vsem):
    cid = lax.axis_index("c")
    sid = lax.axis_index("s")

    pltpu.sync_copy(idx_hbm, idx_v)
    base_s = (cid * NS + sid) * SPW
    pltpu.sync_copy(samp_hbm.at[pl.ds(base_s, SPW)], samp_v)

    def chunk_body(c, _):
        sl = samp_v.at[pl.ds(c * CH, CH)]
        pltpu.sync_copy(tag_hbm.at[sl], t_v)

        def mask_body(i, _):
            t = t_v[pl.ds(i * L, L)]
            s = samp_v[pl.ds(c * CH + i * L, L)]
            inb = (t >= 0) & (t < B)
            tc = jnp.where(inb, t, 0)
            back = plsc.load_gather(idx_v, [tc])
            hit = inb & (back == s)
            tc_v[pl.ds(i * L, L)] = tc
            mf_v[pl.ds(i * L, L)] = jnp.where(hit, 1.0, 0.0).astype(jnp.float32)
            return _

        lax.fori_loop(0, CH // L, mask_body, None)
        gd = pltpu.async_copy(mem_hbm.at[sl], g_v, gsem)
        vd = pltpu.async_copy(val_hbm.at[tc_v], v_v, vsem)
        gd.wait()
        vd.wait()

        def row_body(r, _):
            mrow = plsc.load_gather(mf_v, [jnp.full((L,), r, jnp.int32)])
            for cc in range(D // L):
                g = g_v[r, pl.ds(cc * L, L)]
                v = v_v[r, pl.ds(cc * L, L)]
                g_v[r, pl.ds(cc * L, L)] = g + mrow * (v - g)
            return _

        lax.fori_loop(0, CH, row_body, None)
        pltpu.sync_copy(g_v, out_hbm.at[pl.ds(base_s + c * CH, CH)])
        return _

    lax.fori_loop(0, NCH, chunk_body, None)


@jax.jit
def kernel(mem, idx, val, sample_idx):
    mesh = plsc.VectorSubcoreMesh(
        core_axis_name="c", subcore_axis_name="s",
        num_cores=NC, num_subcores=NS)
    params = pltpu.CompilerParams(
        needs_layout_passes=False, use_tc_tiling_on_sc=False)
    tag_run = pl.kernel(
        _tag_body,
        out_type=jax.ShapeDtypeStruct((MPAD,), jnp.int32),
        mesh=mesh,
        scratch_types=[
            pltpu.VMEM((B,), jnp.int32),
            pltpu.VMEM((RPT,), jnp.int32),
        ],
        compiler_params=params,
    )
    out_run = pl.kernel(
        _out_body,
        out_type=jax.ShapeDtypeStruct((B, D), jnp.float32),
        mesh=mesh,
        scratch_types=[
            pltpu.VMEM((B,), jnp.int32),
            pltpu.VMEM((SPW,), jnp.int32),
            pltpu.VMEM((CH,), jnp.int32),
            pltpu.VMEM((CH,), jnp.int32),
            pltpu.VMEM((CH,), jnp.float32),
            pltpu.VMEM((CH, D), jnp.float32),
            pltpu.VMEM((CH, D), jnp.float32),
            pltpu.SemaphoreType.DMA,
            pltpu.SemaphoreType.DMA,
        ],
        compiler_params=params,
    )
    tag = tag_run(idx)
    return out_run(mem, idx, val, sample_idx, tag)
